# Initial kernel scaffold; baseline (speedup 1.0000x reference)
import jax
import jax.numpy as jnp
from jax.experimental import pallas as pl


def kernel(feat0, feat1, feat2, feat3, boxes0, boxes1):
    raise NotImplementedError("write your pallas kernel here")


# speedup vs baseline: 8.8367x; 8.8367x over previous
import functools

import jax
import jax.numpy as jnp
from jax import lax
from jax.experimental import pallas as pl
from jax.experimental.pallas import tpu as pltpu
from jax.experimental.pallas import tpu_sc as plsc

_H = (25, 50, 100, 200)
_W = (38, 76, 152, 304)
_SCALES = (1.0 / 32.0, 1.0 / 16.0, 1.0 / 8.0, 1.0 / 4.0)
_OFF = (0, 1900, 9500, 39900)
_HW = tuple(h * w for h, w in zip(_H, _W))
_T0 = 384.0 * 384.0
_T1 = 192.0 * 192.0
_T2 = 96.0 * 96.0

_R_PAD = 1024
_BR = 128
_NC, _NS = 2, 16
_NW = _NC * _NS
_RPW = _R_PAD // _NW


def _make_index_body(n0):
    def body(boxes_ref, idx_ref, w_ref):
        b4 = boxes_ref[...]
        x1 = b4[:, 0:1]
        y1 = b4[:, 1:2]
        x2 = b4[:, 2:3]
        y2 = b4[:, 3:4]
        area = (x2 - x1) * (y2 - y1)
        lvl = jnp.where(
            area >= _T0, 0, jnp.where(area >= _T1, 1, jnp.where(area >= _T2, 2, 3))
        ).astype(jnp.int32)

        def sel_f(vals):
            return jnp.where(
                lvl == 0,
                jnp.float32(vals[0]),
                jnp.where(
                    lvl == 1,
                    jnp.float32(vals[1]),
                    jnp.where(lvl == 2, jnp.float32(vals[2]), jnp.float32(vals[3])),
                ),
            )

        def sel_i(vals):
            return jnp.where(
                lvl == 0,
                jnp.int32(vals[0]),
                jnp.where(
                    lvl == 1,
                    jnp.int32(vals[1]),
                    jnp.where(lvl == 2, jnp.int32(vals[2]), jnp.int32(vals[3])),
                ),
            )

        scale = sel_f(_SCALES)
        hf = sel_f([float(h) for h in _H])
        wf = sel_f([float(w) for w in _W])
        hi = sel_i(_H)
        wi = sel_i(_W)
        off = sel_i(_OFF)
        hw = sel_i(_HW)

        rid = pl.program_id(0) * _BR + lax.broadcasted_iota(jnp.int32, (_BR, 1), 0)
        base = off + jnp.where(rid >= n0, hw, 0)

        x1s = x1 * scale
        y1s = y1 * scale
        x2s = x2 * scale
        y2s = y2 * scale
        bw = jnp.maximum(x2s - x1s, 1.0) / 7.0
        bh = jnp.maximum(y2s - y1s, 1.0) / 7.0

        col = lax.broadcasted_iota(jnp.int32, (_BR, 784), 1)
        cx = col & 1
        cy = (col >> 1) & 1
        sx = (col >> 2) & 1
        sy = (col >> 3) & 1
        binc = col >> 4
        by = (binc * 9363) >> 16
        bxn = binc - by * 7

        syk = (by * 2 + sy).astype(jnp.float32)
        sxk = (bxn * 2 + sx).astype(jnp.float32)
        ysv = y1s + (syk + 0.5) / 2.0 * bh
        xsv = x1s + (sxk + 0.5) / 2.0 * bw

        vy = (ysv >= -1.0) & (ysv <= hf)
        vx = (xsv >= -1.0) & (xsv <= wf)
        yc = jnp.clip(ysv, 0.0, hf - 1.0)
        xc = jnp.clip(xsv, 0.0, wf - 1.0)
        y0f = jnp.floor(yc)
        x0f = jnp.floor(xc)
        ly = yc - y0f
        lx = xc - x0f
        y0 = y0f.astype(jnp.int32)
        x0 = x0f.astype(jnp.int32)
        y1i = jnp.minimum(y0 + 1, hi - 1)
        x1i = jnp.minimum(x0 + 1, wi - 1)
        ya = jnp.where(cy == 1, y1i, y0)
        xa = jnp.where(cx == 1, x1i, x0)
        wy = jnp.where(cy == 1, ly, 1.0 - ly)
        wx = jnp.where(cx == 1, lx, 1.0 - lx)
        wgt = 0.25 * wy * wx * jnp.where(vy & vx, 1.0, 0.0)

        idx_ref[...] = base + ya * wi + xa
        w_ref[...] = wgt.astype(jnp.float32)

    return body


def _index_weights(boxes_p, n0):
    return pl.pallas_call(
        _make_index_body(n0),
        grid=(_R_PAD // _BR,),
        in_specs=[pl.BlockSpec((_BR, 4), lambda i: (i, 0))],
        out_specs=[
            pl.BlockSpec((_BR, 784), lambda i: (i, 0)),
            pl.BlockSpec((_BR, 784), lambda i: (i, 0)),
        ],
        out_shape=[
            jax.ShapeDtypeStruct((_R_PAD, 784), jnp.int32),
            jax.ShapeDtypeStruct((_R_PAD, 784), jnp.float32),
        ],
    )(boxes_p)


def _make_sc_kernel():
    mesh = plsc.VectorSubcoreMesh(core_axis_name="c", subcore_axis_name="s")

    @functools.partial(
        pl.kernel,
        mesh=mesh,
        out_type=jax.ShapeDtypeStruct((_R_PAD, 49, 128), jnp.float32),
        scratch_types=[
            pltpu.VMEM((7, 112), jnp.int32),
            pltpu.VMEM((98, 128), jnp.float32),
            pltpu.VMEM((784, 128), jnp.float32),
            pltpu.VMEM((49, 128), jnp.float32),
            pltpu.SemaphoreType.DMA,
        ],
    )
    def sc(idx_hbm, w_hbm, table_hbm, out_hbm, idx_v, w_v, rows_v, out_v, gsem):
        wid = lax.axis_index("s") * _NC + lax.axis_index("c")

        def per_roi(r, carry):
            roi = wid * _RPW + r
            pltpu.sync_copy(idx_hbm.at[roi], idx_v)
            pltpu.sync_copy(w_hbm.at[roi], w_v)
            cps = [
                pltpu.async_copy(
                    table_hbm.at[idx_v.at[j]], rows_v.at[pl.ds(j * 112, 112)], gsem
                )
                for j in range(7)
            ]
            for cp in cps:
                cp.wait()

            def per_bin(b, c2):
                accs = [jnp.zeros((16,), jnp.float32) for _ in range(8)]
                for j in range(16):
                    row = b * 16 + j
                    wv = w_v[2 * b + (j >> 3), pl.ds((j % 8) * 16, 16)]
                    for v in range(8):
                        accs[v] = accs[v] + wv * rows_v[row, pl.ds(v * 16, 16)]
                for v in range(8):
                    out_v[b, pl.ds(v * 16, 16)] = accs[v]
                return c2

            lax.fori_loop(0, 49, per_bin, 0)
            pltpu.sync_copy(out_v, out_hbm.at[roi])
            return carry

        lax.fori_loop(0, _RPW, per_roi, 0)

    return sc


def kernel(feat0, feat1, feat2, feat3, boxes0, boxes1):
    feats = (feat0, feat1, feat2, feat3)
    table = jnp.concatenate(
        [jnp.transpose(f, (0, 2, 3, 1)).reshape(-1, 128) for f in feats], axis=0
    )
    n0 = boxes0.shape[0]
    n = n0 + boxes1.shape[0]
    boxes = jnp.concatenate([boxes0, boxes1], axis=0)
    pad = jnp.broadcast_to(
        jnp.array([0.0, 0.0, 16.0, 16.0], jnp.float32), (_R_PAD - n, 4)
    )
    boxes_p = jnp.concatenate([boxes, pad], axis=0)
    idx, w = _index_weights(boxes_p, n0)
    w3 = jnp.broadcast_to(w[:, :, None], (_R_PAD, 784, 16)).reshape(_R_PAD, 98, 128)
    out = _make_sc_kernel()(idx.reshape(_R_PAD, 7, 112), w3, table)
    out = out[:n].reshape(n, 7, 7, 128)
    return jnp.transpose(out, (0, 3, 1, 2))


# speedup vs baseline: 9.2136x; 1.0427x over previous
import functools

import jax
import jax.numpy as jnp
from jax import lax
from jax.experimental import pallas as pl
from jax.experimental.pallas import tpu as pltpu
from jax.experimental.pallas import tpu_sc as plsc

_H = (25, 50, 100, 200)
_W = (38, 76, 152, 304)
_SCALES = (1.0 / 32.0, 1.0 / 16.0, 1.0 / 8.0, 1.0 / 4.0)
_OFF = (0, 1900, 9500, 39900)
_HW = tuple(h * w for h, w in zip(_H, _W))
_T0 = 384.0 * 384.0
_T1 = 192.0 * 192.0
_T2 = 96.0 * 96.0

_R_PAD = 1024
_BR = 128
_NC, _NS = 2, 16
_NW = _NC * _NS
_RPW = _R_PAD // _NW


def _make_index_body(n0):
    def body(boxes_ref, idx_ref, w_ref):
        b4 = boxes_ref[...]
        x1 = b4[:, 0:1]
        y1 = b4[:, 1:2]
        x2 = b4[:, 2:3]
        y2 = b4[:, 3:4]
        area = (x2 - x1) * (y2 - y1)
        lvl = jnp.where(
            area >= _T0, 0, jnp.where(area >= _T1, 1, jnp.where(area >= _T2, 2, 3))
        ).astype(jnp.int32)

        def sel_f(vals):
            return jnp.where(
                lvl == 0,
                jnp.float32(vals[0]),
                jnp.where(
                    lvl == 1,
                    jnp.float32(vals[1]),
                    jnp.where(lvl == 2, jnp.float32(vals[2]), jnp.float32(vals[3])),
                ),
            )

        def sel_i(vals):
            return jnp.where(
                lvl == 0,
                jnp.int32(vals[0]),
                jnp.where(
                    lvl == 1,
                    jnp.int32(vals[1]),
                    jnp.where(lvl == 2, jnp.int32(vals[2]), jnp.int32(vals[3])),
                ),
            )

        scale = sel_f(_SCALES)
        hf = sel_f([float(h) for h in _H])
        wf = sel_f([float(w) for w in _W])
        hi = sel_i(_H)
        wi = sel_i(_W)
        off = sel_i(_OFF)
        hw = sel_i(_HW)

        rid = pl.program_id(0) * _BR + lax.broadcasted_iota(jnp.int32, (_BR, 1), 0)
        base = off + jnp.where(rid >= n0, hw, 0)

        x1s = x1 * scale
        y1s = y1 * scale
        x2s = x2 * scale
        y2s = y2 * scale
        bw = jnp.maximum(x2s - x1s, 1.0) / 7.0
        bh = jnp.maximum(y2s - y1s, 1.0) / 7.0

        col = lax.broadcasted_iota(jnp.int32, (_BR, 784), 1)
        cx = col & 1
        cy = (col >> 1) & 1
        sx = (col >> 2) & 1
        sy = (col >> 3) & 1
        binc = col >> 4
        by = (binc * 9363) >> 16
        bxn = binc - by * 7

        syk = (by * 2 + sy).astype(jnp.float32)
        sxk = (bxn * 2 + sx).astype(jnp.float32)
        ysv = y1s + (syk + 0.5) / 2.0 * bh
        xsv = x1s + (sxk + 0.5) / 2.0 * bw

        vy = (ysv >= -1.0) & (ysv <= hf)
        vx = (xsv >= -1.0) & (xsv <= wf)
        yc = jnp.clip(ysv, 0.0, hf - 1.0)
        xc = jnp.clip(xsv, 0.0, wf - 1.0)
        y0f = jnp.floor(yc)
        x0f = jnp.floor(xc)
        ly = yc - y0f
        lx = xc - x0f
        y0 = y0f.astype(jnp.int32)
        x0 = x0f.astype(jnp.int32)
        y1i = jnp.minimum(y0 + 1, hi - 1)
        x1i = jnp.minimum(x0 + 1, wi - 1)
        ya = jnp.where(cy == 1, y1i, y0)
        xa = jnp.where(cx == 1, x1i, x0)
        wy = jnp.where(cy == 1, ly, 1.0 - ly)
        wx = jnp.where(cx == 1, lx, 1.0 - lx)
        wgt = 0.25 * wy * wx * jnp.where(vy & vx, 1.0, 0.0)

        idx_ref[...] = base + ya * wi + xa
        w_ref[...] = wgt.astype(jnp.float32)

    return body


def _index_weights(boxes_p, n0):
    return pl.pallas_call(
        _make_index_body(n0),
        grid=(_R_PAD // _BR,),
        in_specs=[pl.BlockSpec((_BR, 4), lambda i: (i, 0))],
        out_specs=[
            pl.BlockSpec((_BR, 784), lambda i: (i, 0)),
            pl.BlockSpec((_BR, 784), lambda i: (i, 0)),
        ],
        out_shape=[
            jax.ShapeDtypeStruct((_R_PAD, 784), jnp.int32),
            jax.ShapeDtypeStruct((_R_PAD, 784), jnp.float32),
        ],
    )(boxes_p)


def _make_sc_kernel():
    mesh = plsc.VectorSubcoreMesh(core_axis_name="c", subcore_axis_name="s")

    @functools.partial(
        pl.kernel,
        mesh=mesh,
        out_type=jax.ShapeDtypeStruct((_R_PAD, 49, 128), jnp.float32),
        scratch_types=[
            pltpu.VMEM((2, 7, 112), jnp.int32),
            pltpu.VMEM((2, 98, 128), jnp.float32),
            pltpu.VMEM((2, 112, 128), jnp.float32),
            pltpu.VMEM((2, 49, 128), jnp.float32),
            pltpu.SemaphoreType.DMA,
            pltpu.SemaphoreType.DMA,
            pltpu.SemaphoreType.DMA,
        ],
    )
    def sc(idx_hbm, w_hbm, table_hbm, out_hbm, idx_v, w_v, rows_v, out_v,
           iwsem, gsem0, gsem1):
        wid = lax.axis_index("s") * _NC + lax.axis_index("c")
        base = wid * _RPW
        gsems = (gsem0, gsem1)

        def start_iw(roi, p):
            pltpu.make_async_copy(idx_hbm.at[roi], idx_v.at[p], iwsem).start()
            pltpu.make_async_copy(w_hbm.at[roi], w_v.at[p], iwsem).start()

        def wait_iw(p):
            pltpu.make_async_copy(idx_hbm.at[0], idx_v.at[p], iwsem).wait()
            pltpu.make_async_copy(w_hbm.at[0], w_v.at[p], iwsem).wait()

        def process(roi, p):
            iv = idx_v.at[p]
            wv_ = w_v.at[p]
            ov = out_v.at[p]
            pltpu.make_async_copy(table_hbm.at[iv.at[0]], rows_v.at[0], gsems[0]).start()
            for c in range(7):
                if c + 1 < 7:
                    pltpu.make_async_copy(
                        table_hbm.at[iv.at[c + 1]], rows_v.at[(c + 1) % 2],
                        gsems[(c + 1) % 2]).start()
                pltpu.make_async_copy(
                    table_hbm.at[iv.at[c]], rows_v.at[c % 2], gsems[c % 2]).wait()
                rb = rows_v.at[c % 2]

                def bin_body(i, acc_c, c=c, rb=rb, wv_=wv_, ov=ov):
                    accs = [jnp.zeros((16,), jnp.float32) for _ in range(8)]
                    for j in range(16):
                        wvec = wv_[14 * c + 2 * i + (j >> 3), pl.ds((j % 8) * 16, 16)]
                        for v in range(8):
                            accs[v] = accs[v] + wvec * rb[i * 16 + j, pl.ds(v * 16, 16)]
                    for v in range(8):
                        ov[7 * c + i, pl.ds(v * 16, 16)] = accs[v]
                    return acc_c

                lax.fori_loop(0, 7, bin_body, 0)
            pltpu.sync_copy(ov, out_hbm.at[roi])

        start_iw(base, 0)

        def pair(k, carry):
            r0 = base + 2 * k
            wait_iw(0)
            start_iw(r0 + 1, 1)
            process(r0, 0)
            wait_iw(1)
            start_iw(jnp.minimum(r0 + 2, _R_PAD - 1), 0)
            process(r0 + 1, 1)
            return carry

        lax.fori_loop(0, _RPW // 2, pair, 0)
        wait_iw(0)

    return sc


def kernel(feat0, feat1, feat2, feat3, boxes0, boxes1):
    feats = (feat0, feat1, feat2, feat3)
    table = jnp.concatenate(
        [jnp.transpose(f, (0, 2, 3, 1)).reshape(-1, 128) for f in feats], axis=0
    )
    n0 = boxes0.shape[0]
    n = n0 + boxes1.shape[0]
    boxes = jnp.concatenate([boxes0, boxes1], axis=0)
    pad = jnp.broadcast_to(
        jnp.array([0.0, 0.0, 16.0, 16.0], jnp.float32), (_R_PAD - n, 4)
    )
    boxes_p = jnp.concatenate([boxes, pad], axis=0)
    idx, w = _index_weights(boxes_p, n0)
    w3 = jnp.broadcast_to(w[:, :, None], (_R_PAD, 784, 16)).reshape(_R_PAD, 98, 128)
    out = _make_sc_kernel()(idx.reshape(_R_PAD, 7, 112), w3, table)
    out = out[:n].reshape(n, 7, 7, 128)
    return jnp.transpose(out, (0, 3, 1, 2))
